# Initial kernel scaffold; baseline (speedup 1.0000x reference)
import jax
import jax.numpy as jnp
from jax.experimental import pallas as pl


def kernel(x, conv1_wm, conv1_bias, s0_c1_wm, s0_c1_bias, s0_c2_wm, s0_c2_bias, s0_c3_wm, s0_c3_bias, s0_cd_wm, s0_cd_bias, s1_c1_wm, s1_c1_bias, s1_c2_wm, s1_c2_bias, s1_c3_wm, s1_c3_bias, s1_cd_wm, s1_cd_bias, s2_c1_wm, s2_c1_bias, s2_c2_wm, s2_c2_bias, s2_c3_wm, s2_c3_bias, s2_cd_wm, s2_cd_bias, s3_c1_wm, s3_c1_bias, s3_c2_wm, s3_c2_bias, s3_c3_wm, s3_c3_bias, s3_cd_wm, s3_cd_bias):
    raise NotImplementedError("write your pallas kernel here")


# speedup vs baseline: 1.1123x; 1.1123x over previous
import jax
import jax.numpy as jnp
from jax.experimental import pallas as pl
from jax.experimental.pallas import tpu as pltpu

_VMEM_LIMIT = 48 * 1024 * 1024
_DN3 = (((2,), (0,)), ((), ()))


def _mm_body_factory(relu, has_res):
    if has_res:
        def body(x_ref, w_ref, b_ref, r_ref, o_ref):
            acc = jnp.dot(x_ref[...], w_ref[...],
                          preferred_element_type=jnp.float32)
            acc = acc + b_ref[...] + r_ref[...].astype(jnp.float32)
            if relu:
                acc = jnp.maximum(acc, 0.0)
            o_ref[...] = acc.astype(o_ref.dtype)
    else:
        def body(x_ref, w_ref, b_ref, o_ref):
            acc = jnp.dot(x_ref[...], w_ref[...],
                          preferred_element_type=jnp.float32)
            acc = acc + b_ref[...]
            if relu:
                acc = jnp.maximum(acc, 0.0)
            o_ref[...] = acc.astype(o_ref.dtype)
    return body


def _mm(x2, wm, bias, relu, residual=None, tm=512):
    M, K = x2.shape
    Cp = wm.shape[1]
    tn = Cp if Cp <= 512 else 512
    tm = min(tm, M)
    grid = (Cp // tn, pl.cdiv(M, tm))
    in_specs = [
        pl.BlockSpec((tm, K), lambda j, i: (i, 0)),
        pl.BlockSpec((K, tn), lambda j, i: (0, j)),
        pl.BlockSpec((1, tn), lambda j, i: (0, j)),
    ]
    args = [x2, wm, bias]
    if residual is not None:
        in_specs.append(pl.BlockSpec((tm, tn), lambda j, i: (i, j)))
        args.append(residual)
    return pl.pallas_call(
        _mm_body_factory(bool(relu), residual is not None),
        out_shape=jax.ShapeDtypeStruct((M, Cp), jnp.bfloat16),
        grid=grid,
        in_specs=in_specs,
        out_specs=pl.BlockSpec((tm, tn), lambda j, i: (i, j)),
        compiler_params=pltpu.CompilerParams(
            dimension_semantics=("parallel", "parallel"),
            vmem_limit_bytes=_VMEM_LIMIT),
    )(*args)


def _c3_body_factory(Ho, C, Cp, tapspec):
    def body(*refs):
        n_in = len(tapspec)
        w_ref, b_ref, o_ref = refs[n_in], refs[n_in + 1], refs[-1]
        acc = jnp.zeros((Ho, Ho, Cp), jnp.float32)
        for x_ref, taps in zip(refs[:n_in], tapspec):
            xb = x_ref[0]
            for t, oi, oj in taps:
                tap = jax.lax.slice(xb, (oi, oj, 0), (oi + Ho, oj + Ho, C))
                acc = acc + jax.lax.dot_general(
                    tap, w_ref[t], _DN3,
                    preferred_element_type=jnp.float32)
        y = jnp.maximum(acc + b_ref[0], 0.0)
        o_ref[0] = y.astype(o_ref.dtype)
    return body


def _conv3x3(x, wm, bias, stride, dil):
    N, H, W, C = x.shape
    Cp = wm.shape[1]
    pad = dil
    Ho = (H + 2 * pad - 2 * dil - 1) // stride + 1
    span = (Ho - 1) * stride + 2 * dil + 1
    pad_hi = span - pad - H
    if stride == 1:
        xp = jnp.pad(x, ((0, 0), (pad, pad_hi), (pad, pad_hi), (0, 0)))
        planes = [xp]
        tapspec = [[(i * 3 + j, i * dil, j * dil)
                    for i in range(3) for j in range(3)]]
    else:
        he = span + (span % 2)
        xp = jnp.pad(x, ((0, 0), (pad, he - pad - H), (pad, he - pad - H),
                         (0, 0)))
        planes, tapspec = [], []
        for pi in range(2):
            for pj in range(2):
                planes.append(xp[:, pi::2, pj::2, :])
                tapspec.append([(i * 3 + j, i // 2, j // 2)
                                for i in range(pi, 3, 2)
                                for j in range(pj, 3, 2)])
    Hp = planes[0].shape[1]
    w9 = wm.reshape(9, C, Cp)
    b3 = bias.reshape(1, 1, Cp)
    in_specs = [pl.BlockSpec((1, Hp, Hp, C), lambda n: (n, 0, 0, 0))
                for _ in planes]
    in_specs.append(pl.BlockSpec((9, C, Cp), lambda n: (0, 0, 0)))
    in_specs.append(pl.BlockSpec((1, 1, Cp), lambda n: (0, 0, 0)))
    return pl.pallas_call(
        _c3_body_factory(Ho, C, Cp, tapspec),
        out_shape=jax.ShapeDtypeStruct((N, Ho, Ho, Cp), jnp.bfloat16),
        grid=(N,),
        in_specs=in_specs,
        out_specs=pl.BlockSpec((1, Ho, Ho, Cp), lambda n: (n, 0, 0, 0)),
        compiler_params=pltpu.CompilerParams(
            dimension_semantics=("parallel",),
            vmem_limit_bytes=_VMEM_LIMIT),
    )(*planes, w9, b3)


def _mp_body_factory(Ho, C, tapspec):
    def body(*refs):
        o_ref = refs[-1]
        m = None
        for x_ref, taps in zip(refs[:-1], tapspec):
            xb = x_ref[0]
            for _, oi, oj in taps:
                tap = jax.lax.slice(xb, (oi, oj, 0), (oi + Ho, oj + Ho, C))
                m = tap if m is None else jnp.maximum(m, tap)
        o_ref[0] = m
    return body


def _maxpool(x):
    N, H, W, C = x.shape
    Ho = -(-(H - 1) // 2) + 1
    if (Ho - 1) * 2 >= H + 1:
        Ho -= 1
    span = (Ho - 1) * 2 + 3
    he = span + (span % 2)
    neg = float(jnp.finfo(x.dtype).min)
    xp = jnp.pad(x, ((0, 0), (1, he - 1 - H), (1, he - 1 - H), (0, 0)),
                 constant_values=neg)
    planes, tapspec = [], []
    for pi in range(2):
        for pj in range(2):
            planes.append(xp[:, pi::2, pj::2, :])
            tapspec.append([(0, i // 2, j // 2)
                            for i in range(pi, 3, 2)
                            for j in range(pj, 3, 2)])
    Hp = planes[0].shape[1]
    return pl.pallas_call(
        _mp_body_factory(Ho, C, tapspec),
        out_shape=jax.ShapeDtypeStruct((N, Ho, Ho, C), x.dtype),
        grid=(N,),
        in_specs=[pl.BlockSpec((1, Hp, Hp, C), lambda n: (n, 0, 0, 0))
                  for _ in planes],
        out_specs=pl.BlockSpec((1, Ho, Ho, C), lambda n: (n, 0, 0, 0)),
        compiler_params=pltpu.CompilerParams(
            dimension_semantics=("parallel",),
            vmem_limit_bytes=_VMEM_LIMIT),
    )(*planes)


def _conv1(x, wm, bias):
    N, H, W, C = x.shape
    s, pad, k = 2, 3, 7
    Ho = (H + 2 * pad - k) // s + 1
    xp = jnp.pad(x, ((0, 0), (pad, pad), (pad, pad), (0, 0)))
    L = (Ho - 1) * s + 1
    cols = [xp[:, i:i + L:s, j:j + L:s, :] for i in range(k) for j in range(k)]
    patches = jnp.concatenate(cols, axis=-1).reshape(N * Ho * Ho, k * k * C)
    K = patches.shape[1]
    out = _mm(patches, wm[:K], bias, relu=True)
    return out.reshape(N, Ho, Ho, wm.shape[1])


def _bottleneck(x, stride, dil, c1w, c1b, c2w, c2b, c3w, c3b, cdw, cdb):
    N, H, W, C = x.shape
    xs = x if stride == 1 else x[:, ::stride, ::stride, :]
    ident = _mm(xs.reshape(-1, C), cdw, cdb, relu=False)
    h1 = _mm(x.reshape(-1, C), c1w, c1b, relu=True)
    h1 = h1.reshape(N, H, W, c1w.shape[1])
    h2 = _conv3x3(h1, c2w, c2b, stride, dil)
    Ho = h2.shape[1]
    h3 = _mm(h2.reshape(N * Ho * Ho, c2w.shape[1]), c3w, c3b,
             relu=True, residual=ident)
    return h3.reshape(N, Ho, Ho, c3w.shape[1])


def kernel(x, conv1_wm, conv1_bias,
           s0_c1_wm, s0_c1_bias, s0_c2_wm, s0_c2_bias,
           s0_c3_wm, s0_c3_bias, s0_cd_wm, s0_cd_bias,
           s1_c1_wm, s1_c1_bias, s1_c2_wm, s1_c2_bias,
           s1_c3_wm, s1_c3_bias, s1_cd_wm, s1_cd_bias,
           s2_c1_wm, s2_c1_bias, s2_c2_wm, s2_c2_bias,
           s2_c3_wm, s2_c3_bias, s2_cd_wm, s2_cd_bias,
           s3_c1_wm, s3_c1_bias, s3_c2_wm, s3_c2_bias,
           s3_c3_wm, s3_c3_bias, s3_cd_wm, s3_cd_bias):
    xh = jnp.transpose(x, (0, 2, 3, 1)).astype(jnp.bfloat16)
    y = _conv1(xh, conv1_wm, conv1_bias)
    y = _maxpool(y)
    f0 = _bottleneck(y, 1, 1, s0_c1_wm, s0_c1_bias, s0_c2_wm, s0_c2_bias,
                     s0_c3_wm, s0_c3_bias, s0_cd_wm, s0_cd_bias)
    f1 = _bottleneck(f0, 2, 1, s1_c1_wm, s1_c1_bias, s1_c2_wm, s1_c2_bias,
                     s1_c3_wm, s1_c3_bias, s1_cd_wm, s1_cd_bias)
    f2 = _bottleneck(f1, 2, 1, s2_c1_wm, s2_c1_bias, s2_c2_wm, s2_c2_bias,
                     s2_c3_wm, s2_c3_bias, s2_cd_wm, s2_cd_bias)
    f3 = _bottleneck(f2, 1, 2, s3_c1_wm, s3_c1_bias, s3_c2_wm, s3_c2_bias,
                     s3_c3_wm, s3_c3_bias, s3_cd_wm, s3_cd_bias)
    return [jnp.transpose(f, (0, 3, 1, 2)).astype(jnp.float32)
            for f in (f0, f1, f2, f3)]
